# baseline (speedup 1.0000x reference)
import functools

import jax
import jax.numpy as jnp
from jax import lax
from jax.experimental import pallas as pl
from jax.experimental.pallas import tpu as pltpu
from jax.experimental.pallas import tpu_sc as plsc

F = 26
V1 = 100001
D = 32
B = 16384
N = B * F

NC = 2
NS = 16
L = 16
NW = NC * NS

N_PER_W = N // NW
CHUNK = 1024
N_CHUNKS = N_PER_W // CHUNK
IDX_ROWS = CHUNK // 128


def _body(x_hbm, tab_hbm, out_hbm, xv, idxv, rows, sem):
    wid = lax.axis_index("s") * NC + lax.axis_index("c")
    base = wid * N_PER_W

    def chunk_body(g, _):
        start = base + g * CHUNK
        pltpu.sync_copy(x_hbm.at[pl.ds(start, CHUNK)], xv)

        def vec_body(i, _):
            pos = start + i * L + lax.iota(jnp.int32, 16)
            offs = lax.rem(pos, F) * V1
            j = i // (128 // L)
            col = (i % (128 // L)) * L
            idxv[j, pl.ds(col, L)] = xv[pl.ds(i * L, L)] + offs
            return 0

        lax.fori_loop(0, CHUNK // L, vec_body, 0)

        copies = [
            pltpu.async_copy(tab_hbm.at[idxv.at[j]],
                             rows.at[pl.ds(j * 128, 128)], sem)
            for j in range(IDX_ROWS)
        ]
        for c in copies:
            c.wait()

        pltpu.sync_copy(rows, out_hbm.at[pl.ds(start, CHUNK)])
        return 0

    lax.fori_loop(0, N_CHUNKS, chunk_body, 0)


@jax.jit
def kernel(x, tables):
    x_flat = x.reshape(N)
    tab_flat = tables.reshape(F * V1, D)
    mesh = plsc.VectorSubcoreMesh(core_axis_name="c", subcore_axis_name="s",
                                  num_cores=NC, num_subcores=NS)
    out = pl.kernel(
        _body,
        out_type=jax.ShapeDtypeStruct((N, D), jnp.float32),
        mesh=mesh,
        scratch_types=[
            pltpu.VMEM((CHUNK,), jnp.int32),
            pltpu.VMEM((IDX_ROWS, 128), jnp.int32),
            pltpu.VMEM((CHUNK, D), jnp.float32),
            pltpu.SemaphoreType.DMA,
        ],
        compiler_params=pltpu.CompilerParams(use_tc_tiling_on_sc=False),
    )(x_flat, tab_flat)
    return out.reshape(B, F, D)


# speedup vs baseline: 31.3901x; 31.3901x over previous
import jax
import jax.numpy as jnp
from jax import lax
from jax.experimental import pallas as pl
from jax.experimental.pallas import tpu as pltpu
from jax.experimental.pallas import tpu_sc as plsc

F = 26
V1 = 100001
D = 32
B = 16384

NC = 2
NS = 16
L = 16
NW = NC * NS

HALF = B // 2


def _body(xt_hbm, tabt_hbm, out_hbm, idxv, rowv, outv):
    d = lax.axis_index("s") * NC + lax.axis_index("c")

    def f_body(f, _):
        pltpu.sync_copy(xt_hbm.at[f], idxv)
        pltpu.sync_copy(tabt_hbm.at[f, d], rowv)

        for h in range(2):
            def gbody(i, _):
                idxs = idxv[pl.ds(h * HALF + i * L, L)]
                outv[pl.ds(i * L, L)] = plsc.load_gather(rowv, [idxs])
                return 0

            lax.fori_loop(0, HALF // L, gbody, 0)
            pltpu.sync_copy(outv, out_hbm.at[f, d, pl.ds(h * HALF, HALF)])
        return 0

    lax.fori_loop(0, F, f_body, 0)


@jax.jit
def kernel(x, tables):
    x_t = x.T
    tab_t = jnp.transpose(tables, (0, 2, 1))
    mesh = plsc.VectorSubcoreMesh(core_axis_name="c", subcore_axis_name="s",
                                  num_cores=NC, num_subcores=NS)
    out_t = pl.kernel(
        _body,
        out_type=jax.ShapeDtypeStruct((F, D, B), jnp.float32),
        mesh=mesh,
        scratch_types=[
            pltpu.VMEM((B,), jnp.int32),
            pltpu.VMEM((V1,), jnp.float32),
            pltpu.VMEM((HALF,), jnp.float32),
        ],
        compiler_params=pltpu.CompilerParams(use_tc_tiling_on_sc=True,
                                             needs_layout_passes=False),
    )(x_t, tab_t)
    return jnp.transpose(out_t, (2, 0, 1))


# speedup vs baseline: 55.4112x; 1.7652x over previous
import jax
import jax.numpy as jnp
from jax import lax
from jax.experimental import pallas as pl
from jax.experimental.pallas import tpu as pltpu
from jax.experimental.pallas import tpu_sc as plsc

F = 26
V1 = 100001
D = 32
B = 16384

NC = 2
NS = 16
L = 16
NW = NC * NS

CB = 4096
U = 8


def _body(xt_hbm, tabt_hbm, out_hbm, idxv, rowv, ob0, ob1,
          semr, semi, semo0, semo1):
    d = lax.axis_index("s") * NC + lax.axis_index("c")
    obs = (ob0, ob1)
    sems = (semo0, semo1)

    def f_body(f, _):
        cr = pltpu.async_copy(tabt_hbm.at[f, d], rowv, semr)
        ci = pltpu.async_copy(xt_hbm.at[f], idxv, semi)
        ci.wait()
        cr.wait()

        copies = [None, None, None, None]
        for h in range(4):
            ob = obs[h % 2]
            if h >= 2:
                copies[h - 2].wait()

            def gstep(i, _, h=h, ob=ob):
                base = h * CB + i * (U * L)
                idxs = [idxv[pl.ds(base + u * L, L)] for u in range(U)]
                vals = [plsc.load_gather(rowv, [ix]) for ix in idxs]
                for u in range(U):
                    ob[pl.ds(i * (U * L) + u * L, L)] = vals[u]
                return 0

            lax.fori_loop(0, CB // (U * L), gstep, 0)
            copies[h] = pltpu.async_copy(
                ob, out_hbm.at[f, d, pl.ds(h * CB, CB)], sems[h % 2])
        copies[2].wait()
        copies[3].wait()
        return 0

    lax.fori_loop(0, F, f_body, 0)


@jax.jit
def kernel(x, tables):
    x_t = x.T
    tab_t = jnp.transpose(tables, (0, 2, 1))
    mesh = plsc.VectorSubcoreMesh(core_axis_name="c", subcore_axis_name="s",
                                  num_cores=NC, num_subcores=NS)
    out_t = pl.kernel(
        _body,
        out_type=jax.ShapeDtypeStruct((F, D, B), jnp.float32),
        mesh=mesh,
        scratch_types=[
            pltpu.VMEM((B,), jnp.int32),
            pltpu.VMEM((V1,), jnp.float32),
            pltpu.VMEM((CB,), jnp.float32),
            pltpu.VMEM((CB,), jnp.float32),
            pltpu.SemaphoreType.DMA,
            pltpu.SemaphoreType.DMA,
            pltpu.SemaphoreType.DMA,
            pltpu.SemaphoreType.DMA,
        ],
        compiler_params=pltpu.CompilerParams(use_tc_tiling_on_sc=True,
                                             needs_layout_passes=False),
    )(x_t, tab_t)
    return jnp.transpose(out_t, (2, 0, 1))


# speedup vs baseline: 56.1476x; 1.0133x over previous
import jax
import jax.numpy as jnp
from jax import lax
from jax.experimental import pallas as pl
from jax.experimental.pallas import tpu as pltpu
from jax.experimental.pallas import tpu_sc as plsc

F = 26
V1 = 100001
D = 32
B = 16384

NC = 2
NS = 16
L = 16
NW = NC * NS

CB = 4096
U = 8


def _body(xt_hbm, tabt_hbm, out_hbm, idxv, rowv, ob0, ob1,
          semr, semi, semo0, semo1):
    s = lax.axis_index("s")
    d = s * NC + lax.axis_index("c")
    obs = (ob0, ob1)
    sems = (semo0, semo1)

    def f_body(f, _):
        cr = pltpu.async_copy(tabt_hbm.at[f, d], rowv, semr)
        ci = pltpu.async_copy(xt_hbm.at[f], idxv, semi)
        ci.wait()
        cr.wait()

        for h in range(4):
            ob = obs[h % 2]
            g = f * 4 + h

            @pl.when(g >= 2)
            def _(ob=ob, h=h):
                pltpu.make_async_copy(
                    ob, out_hbm.at[f, d, pl.ds(h * CB, CB)],
                    sems[h % 2]).wait()

            def gstep(i, _, h=h, ob=ob):
                base = h * CB + i * (U * L)
                idxs = [idxv[pl.ds(base + u * L, L)] for u in range(U)]
                vals = [plsc.load_gather(rowv, [ix]) for ix in idxs]
                for u in range(U):
                    ob[pl.ds(i * (U * L) + u * L, L)] = vals[u]
                return 0

            lax.fori_loop(0, CB // (U * L), gstep, 0)
            pltpu.async_copy(
                ob, out_hbm.at[f, d, pl.ds(h * CB, CB)], sems[h % 2])
        return 0

    lax.fori_loop(0, F, f_body, 0)
    for h in (2, 3):
        pltpu.make_async_copy(
            obs[h % 2], out_hbm.at[F - 1, d, pl.ds(h * CB, CB)],
            sems[h % 2]).wait()


@jax.jit
def kernel(x, tables):
    x_t = x.T
    tab_t = jnp.transpose(tables, (0, 2, 1))
    mesh = plsc.VectorSubcoreMesh(core_axis_name="c", subcore_axis_name="s",
                                  num_cores=NC, num_subcores=NS)
    out_t = pl.kernel(
        _body,
        out_type=jax.ShapeDtypeStruct((F, D, B), jnp.float32),
        mesh=mesh,
        scratch_types=[
            pltpu.VMEM((B,), jnp.int32),
            pltpu.VMEM((V1,), jnp.float32),
            pltpu.VMEM((CB,), jnp.float32),
            pltpu.VMEM((CB,), jnp.float32),
            pltpu.SemaphoreType.DMA,
            pltpu.SemaphoreType.DMA,
            pltpu.SemaphoreType.DMA,
            pltpu.SemaphoreType.DMA,
        ],
        compiler_params=pltpu.CompilerParams(use_tc_tiling_on_sc=True,
                                             needs_layout_passes=False),
    )(x_t, tab_t)
    return jnp.transpose(out_t, (2, 0, 1))


# speedup vs baseline: 57.6207x; 1.0262x over previous
import jax
import jax.numpy as jnp
from jax import lax
from jax.experimental import pallas as pl
from jax.experimental.pallas import tpu as pltpu
from jax.experimental.pallas import tpu_sc as plsc

F = 26
V1 = 100001
D = 32
B = 16384

NC = 2
NS = 16
L = 16
NW = NC * NS

CB = 4096
U = 8
FH = 13


def _body(xt_hbm, tabt_hbm, out_hbm, ib0, ib1, rowv, ob0, ob1, sidx,
          semr, semi0, semi1, semo0, semo1):
    s = lax.axis_index("s")
    d = s * NC + lax.axis_index("c")
    obs = (ob0, ob1)
    ibs = (ib0, ib1)
    semo = (semo0, semo1)
    semi = (semi0, semi1)

    def stage(base):
        @pl.when(s < FH)
        def _():
            pltpu.sync_copy(xt_hbm.at[base + s], sidx.at[s])

    def mega(base):
        def f_body(k, _):
            f = base + k
            cr = pltpu.async_copy(tabt_hbm.at[f, d], rowv, semr)
            cis = [None] * 4
            for h in (0, 1):
                cis[h] = pltpu.async_copy(
                    xt_hbm.at[f, pl.ds(h * CB, CB)], ibs[h], semi[h])
            cr.wait()

            for h in range(4):
                ob = obs[h % 2]
                ib = ibs[h % 2]
                cis[h].wait()

                @pl.when(f * 4 + h >= 2)
                def _(ob=ob, h=h):
                    pltpu.make_async_copy(
                        ob, out_hbm.at[f, d, pl.ds(h * CB, CB)],
                        semo[h % 2]).wait()

                def gstep(i, _, ib=ib, ob=ob):
                    base_i = i * (U * L)
                    idxs = [ib[pl.ds(base_i + u * L, L)] for u in range(U)]
                    vals = [plsc.load_gather(rowv, [ix]) for ix in idxs]
                    for u in range(U):
                        ob[pl.ds(base_i + u * L, L)] = vals[u]
                    return 0

                lax.fori_loop(0, CB // (U * L), gstep, 0)
                pltpu.async_copy(
                    ob, out_hbm.at[f, d, pl.ds(h * CB, CB)], semo[h % 2])
                if h + 2 < 4:
                    cis[h + 2] = pltpu.async_copy(
                        xt_hbm.at[f, pl.ds((h + 2) * CB, CB)],
                        ibs[h % 2], semi[h % 2])
            return 0

        lax.fori_loop(0, FH, f_body, 0)

    mega(0)
    mega(FH)
    for h in (2, 3):
        pltpu.make_async_copy(
            obs[h % 2], out_hbm.at[F - 1, d, pl.ds(h * CB, CB)],
            semo[h % 2]).wait()


@jax.jit
def kernel(x, tables):
    x_t = x.T
    tab_t = jnp.transpose(tables, (0, 2, 1))
    mesh = plsc.VectorSubcoreMesh(core_axis_name="c", subcore_axis_name="s",
                                  num_cores=NC, num_subcores=NS)
    out_t = pl.kernel(
        _body,
        out_type=jax.ShapeDtypeStruct((F, D, B), jnp.float32),
        mesh=mesh,
        scratch_types=[
            pltpu.VMEM((CB,), jnp.int32),
            pltpu.VMEM((CB,), jnp.int32),
            pltpu.VMEM((V1,), jnp.float32),
            pltpu.VMEM((CB,), jnp.float32),
            pltpu.VMEM((CB,), jnp.float32),
            pltpu.VMEM_SHARED((FH, B), jnp.int32),
            pltpu.SemaphoreType.DMA,
            pltpu.SemaphoreType.DMA,
            pltpu.SemaphoreType.DMA,
            pltpu.SemaphoreType.DMA,
            pltpu.SemaphoreType.DMA,
        ],
        compiler_params=pltpu.CompilerParams(use_tc_tiling_on_sc=True,
                                             needs_layout_passes=False),
    )(x_t, tab_t)
    return jnp.transpose(out_t, (2, 0, 1))
